# Initial kernel scaffold; baseline (speedup 1.0000x reference)
import functools
import math

import numpy as np
import jax
import jax.numpy as jnp
from jax import lax
from jax.experimental import pallas as pl
from jax.experimental.pallas import tpu as pltpu
from jax.experimental.pallas import tpu_sc as plsc

_B = 2
_X, _Y, _Z = 200, 200, 16
_N = _X * _Y * _Z
_C = 18
_FREE = 17
_EPS = 1e-06
_NUM_RAYS = 2000
_MAX_DIST = 60.0
_VOXEL = 0.4
_PC_MIN = np.array([-40.0, -40.0, -1.0], dtype=np.float32)
_ORIGIN = np.array([0.9858, 0.0, 1.8402], dtype=np.float32)
_GRID = np.array([_X, _Y, _Z], dtype=np.int32)

_NC = 2
_NS = 16
_NW = _NC * _NS
_LANES = 16
_GPW = 8
_RPW = _GPW * _LANES
_DUMMY = _B * _N


def _ray_constants():
    pitch_angles = []
    for k in range(10):
        pitch_angles.append(-(math.pi / 2 - math.atan(k + 1)))
    while pitch_angles[-1] < 0.21:
        delta = pitch_angles[-1] - pitch_angles[-2]
        pitch_angles.append(pitch_angles[-1] + delta)
    rays = []
    for pitch in pitch_angles:
        for az_deg in range(360):
            az = math.radians(az_deg)
            rays.append((math.cos(pitch) * math.cos(az),
                         math.cos(pitch) * math.sin(az),
                         math.sin(pitch)))
    rays = np.array(rays, dtype=np.float32)
    dt = (_VOXEL / (np.abs(rays).max(axis=1) + 1e-08)).astype(np.float32)
    s_max = int(min(np.ceil(_MAX_DIST / dt.min()), 512))
    sel = np.linspace(0, rays.shape[0] - 1, _NUM_RAYS).astype(np.int64)
    dirs = rays[sel]
    dts = dt[sel]
    t = (np.arange(s_max, dtype=np.float32)[None, :] + 0.5) * dts[:, None]
    pts = _ORIGIN[None, None, :] + dirs[:, None, :] * t[:, :, None]
    vox = np.floor((pts - _PC_MIN[None, None, :]) / _VOXEL).astype(np.int32)
    in_range = (np.all((vox >= 0) & (vox < _GRID[None, None, :]), axis=-1)
                & (t <= _MAX_DIST))
    voxc = np.clip(vox, 0, _GRID[None, None, :] - 1)
    flat = (voxc[..., 0] * (_Y * _Z) + voxc[..., 1] * _Z + voxc[..., 2])
    return flat.astype(np.int64), in_range, s_max


def _build_tables():
    flat, in_range, s_max = _ray_constants()
    e = in_range.sum(axis=1).astype(np.int32)
    assert np.array_equal(in_range, np.arange(s_max)[None, :] < e[:, None])
    s_uni = int(-(-int(e.max()) // 8) * 8)
    rtot = _NW * _RPW
    idx_full = np.full((rtot, s_uni), _DUMMY, dtype=np.int64)
    smask = np.arange(s_uni)[None, :] < e[:, None]
    base = flat[:, :s_uni]
    for b in range(_B):
        blk = idx_full[b * _NUM_RAYS:(b + 1) * _NUM_RAYS]
        blk[smask] = (b * _N + base)[smask]
    e_full = np.zeros((rtot,), dtype=np.int32)
    e_full[:_NUM_RAYS] = e
    e_full[_NUM_RAYS:2 * _NUM_RAYS] = e
    idx_v = idx_full.reshape(_NW, _GPW, _LANES, s_uni).transpose(0, 1, 3, 2)
    n_chunk = _GPW * s_uni * _LANES // 128
    idx_chunks = np.ascontiguousarray(idx_v).reshape(_NW, n_chunk, 128)
    e_tbl = e_full.reshape(_NW, _RPW)
    return idx_chunks.astype(np.int32), e_tbl, s_uni, n_chunk


_IDX_NP, _E_NP, _SUNI, _NCHUNK = _build_tables()
_GSTRIDE = _SUNI * _LANES

_LB = 12800
_NBLK = _B * _N // _LB


def _table_kernel(lt_ref, gt_ref, mk_ref, out_ref):
    l = lt_ref[...]
    gt = gt_ref[0]
    mk = mk_ref[0]
    m = jnp.max(l, axis=0, keepdims=True)
    se = jnp.sum(jnp.exp(l - m), axis=0, keepdims=True)
    lse = jnp.log(se) + m
    cio = lax.broadcasted_iota(jnp.int32, (_C, _LB), 0)
    lfree = jnp.sum(jnp.where(cio == _FREE, l, 0.0), axis=0, keepdims=True)
    lgt = jnp.sum(jnp.where(cio == gt, l, 0.0), axis=0, keepdims=True)
    p = jnp.clip(1.0 - jnp.exp(lfree - lse), _EPS, 1.0 - _EPS)
    pre = -jnp.log(1.0 - p)
    hit = -jnp.log(p) + (lse - lgt)
    occf = (gt != _FREE) & (mk > 0)
    w = jnp.where(occf, jnp.maximum(hit, 1e-30), -jnp.maximum(pre, 1e-30))
    out_ref[0] = w


def _table_call(lt, gt3, mk3):
    return pl.pallas_call(
        _table_kernel,
        grid=(_NBLK,),
        in_specs=[
            pl.BlockSpec((_C, _LB), lambda i: (0, i)),
            pl.BlockSpec((1, 1, _LB), lambda i: (i, 0, 0)),
            pl.BlockSpec((1, 1, _LB), lambda i: (i, 0, 0)),
        ],
        out_specs=pl.BlockSpec((1, 1, _LB), lambda i: (i, 0, 0)),
        out_shape=jax.ShapeDtypeStruct((_NBLK, 1, _LB), jnp.float32),
    )(lt, gt3, mk3)


_MESH = plsc.VectorSubcoreMesh(core_axis_name="c", subcore_axis_name="s",
                               num_cores=_NC, num_subcores=_NS)


@functools.partial(
    pl.kernel,
    out_type=jax.ShapeDtypeStruct((_NW, 4, _LANES), jnp.float32),
    mesh=_MESH,
    scratch_types=[
        pltpu.VMEM((_NCHUNK, 128), jnp.int32),
        pltpu.VMEM((_NCHUNK * 128,), jnp.float32),
        pltpu.VMEM((_RPW,), jnp.int32),
        pltpu.VMEM((4, _LANES), jnp.float32),
        pltpu.SemaphoreType.DMA,
    ],
)
def _scan_kernel(w_hbm, idx_hbm, e_hbm, out_hbm, idx_v, vals_v, e_v, acc_v,
                 sem):
    wid = lax.axis_index("s") * _NC + lax.axis_index("c")
    pltpu.sync_copy(idx_hbm.at[wid], idx_v)
    pltpu.sync_copy(e_hbm.at[wid], e_v)

    def fire(j, carry):
        pltpu.async_copy(w_hbm.at[idx_v.at[j]],
                         vals_v.at[pl.ds(j * 128, 128)], sem)
        return carry

    lax.fori_loop(0, _NCHUNK, fire, 0, unroll=False)

    def drain(j, carry):
        pltpu.make_async_copy(w_hbm.at[idx_v.at[j]],
                              vals_v.at[pl.ds(j * 128, 128)], sem).wait()
        return carry

    lax.fori_loop(0, _NCHUNK, drain, 0, unroll=False)

    zeros = jnp.zeros((_LANES,), jnp.float32)
    a_pre, a_prec, a_hit, a_hitc = zeros, zeros, zeros, zeros
    for g in range(_GPW):
        e_g = e_v[pl.ds(g * _LANES, _LANES)]

        def cond(c, e_g=e_g):
            s, hitv = c[0], c[1]
            return jnp.any(jnp.logical_not(hitv) & (s < e_g))

        def body(c, g=g, e_g=e_g):
            s, hitv, pv, pc, ap, apc, ah, ahc = c
            wv = vals_v[pl.ds(g * _GSTRIDE + s * _LANES, _LANES)]
            alive = jnp.logical_not(hitv) & (s < e_g)
            occ = wv > 0.0
            take = occ & alive
            ap = ap + jnp.where(take, pv, 0.0)
            apc = apc + jnp.where(take, pc, 0.0)
            ah = ah + jnp.where(take, wv, 0.0)
            ahc = ahc + jnp.where(take, 1.0, 0.0)
            hitv = hitv | take
            upd = alive & jnp.logical_not(occ)
            pv = pv + jnp.where(upd, -wv, 0.0)
            pc = pc + jnp.where(upd, 1.0, 0.0)
            return (s + 1, hitv, pv, pc, ap, apc, ah, ahc)

        init = (jnp.int32(0), jnp.zeros((_LANES,), jnp.bool_), zeros, zeros,
                a_pre, a_prec, a_hit, a_hitc)
        res = lax.while_loop(cond, body, init)
        a_pre, a_prec, a_hit, a_hitc = res[4], res[5], res[6], res[7]

    acc_v[0] = a_pre
    acc_v[1] = a_prec
    acc_v[2] = a_hit
    acc_v[3] = a_hitc
    pltpu.sync_copy(acc_v, out_hbm.at[wid])


def kernel(occ_logits, voxel_semantics, mask_camera):
    lt = jnp.transpose(occ_logits.reshape(_B * _N, _C))
    gt3 = voxel_semantics.reshape(_NBLK, 1, _LB)
    mk3 = mask_camera.reshape(_NBLK, 1, _LB)
    tbl = _table_call(lt, gt3, mk3).reshape(_B * _N)
    w_full = jnp.concatenate(
        [tbl, jnp.full((8,), -1e-30, dtype=jnp.float32)])
    parts = _scan_kernel(w_full, _IDX_NP, _E_NP)
    sums = jnp.sum(parts, axis=(0, 2))
    loss = (sums[0] / jnp.maximum(sums[1], 1.0)
            + sums[2] / jnp.maximum(sums[3], 1.0))
    return loss


# speedup vs baseline: 4.6600x; 4.6600x over previous
import functools
import math

import numpy as np
import jax
import jax.numpy as jnp
from jax import lax
from jax.experimental import pallas as pl
from jax.experimental.pallas import tpu as pltpu
from jax.experimental.pallas import tpu_sc as plsc

_B = 2
_X, _Y, _Z = 200, 200, 16
_N = _X * _Y * _Z
_C = 18
_FREE = 17
_EPS = 1e-06
_NUM_RAYS = 2000
_MAX_DIST = 60.0
_VOXEL = 0.4
_PC_MIN = np.array([-40.0, -40.0, -1.0], dtype=np.float32)
_ORIGIN = np.array([0.9858, 0.0, 1.8402], dtype=np.float32)
_GRID = np.array([_X, _Y, _Z], dtype=np.int32)

_NC = 2
_NS = 16
_NW = _NC * _NS
_LANES = 16
_GPW = 8
_RPW = _GPW * _LANES
_DUMMY = _B * _N


def _ray_constants():
    pitch_angles = []
    for k in range(10):
        pitch_angles.append(-(math.pi / 2 - math.atan(k + 1)))
    while pitch_angles[-1] < 0.21:
        delta = pitch_angles[-1] - pitch_angles[-2]
        pitch_angles.append(pitch_angles[-1] + delta)
    rays = []
    for pitch in pitch_angles:
        for az_deg in range(360):
            az = math.radians(az_deg)
            rays.append((math.cos(pitch) * math.cos(az),
                         math.cos(pitch) * math.sin(az),
                         math.sin(pitch)))
    rays = np.array(rays, dtype=np.float32)
    dt = (_VOXEL / (np.abs(rays).max(axis=1) + 1e-08)).astype(np.float32)
    s_max = int(min(np.ceil(_MAX_DIST / dt.min()), 512))
    sel = np.linspace(0, rays.shape[0] - 1, _NUM_RAYS).astype(np.int64)
    dirs = rays[sel]
    dts = dt[sel]
    t = (np.arange(s_max, dtype=np.float32)[None, :] + 0.5) * dts[:, None]
    pts = _ORIGIN[None, None, :] + dirs[:, None, :] * t[:, :, None]
    vox = np.floor((pts - _PC_MIN[None, None, :]) / _VOXEL).astype(np.int32)
    in_range = (np.all((vox >= 0) & (vox < _GRID[None, None, :]), axis=-1)
                & (t <= _MAX_DIST))
    voxc = np.clip(vox, 0, _GRID[None, None, :] - 1)
    flat = (voxc[..., 0] * (_Y * _Z) + voxc[..., 1] * _Z + voxc[..., 2])
    return flat.astype(np.int64), in_range, s_max


def _build_tables():
    flat, in_range, s_max = _ray_constants()
    e = in_range.sum(axis=1).astype(np.int32)
    assert np.array_equal(in_range, np.arange(s_max)[None, :] < e[:, None])
    s_uni = int(-(-int(e.max()) // 8) * 8)
    rtot = _NW * _RPW
    idx_full = np.full((rtot, s_uni), _DUMMY, dtype=np.int64)
    smask = np.arange(s_uni)[None, :] < e[:, None]
    base = flat[:, :s_uni]
    for b in range(_B):
        blk = idx_full[b * _NUM_RAYS:(b + 1) * _NUM_RAYS]
        blk[smask] = (b * _N + base)[smask]
    e_full = np.zeros((rtot,), dtype=np.int32)
    e_full[:_NUM_RAYS] = e
    e_full[_NUM_RAYS:2 * _NUM_RAYS] = e
    idx_v = idx_full.reshape(_NW, _GPW, _LANES, s_uni).transpose(0, 1, 3, 2)
    n_chunk = _GPW * s_uni * _LANES // 128
    idx_chunks = np.ascontiguousarray(idx_v).reshape(_NW, n_chunk, 128)
    e_tbl = e_full.reshape(_NW, _RPW)
    return idx_chunks.astype(np.int32), e_tbl, s_uni, n_chunk


_IDX_NP, _E_NP, _SUNI, _NCHUNK = _build_tables()
_GSTRIDE = _SUNI * _LANES

_LB = 12800
_NBLK = _B * _N // _LB


def _table_kernel(lt_ref, gt_ref, mk_ref, out_ref):
    l = lt_ref[...]
    gt = gt_ref[0]
    mk = mk_ref[0]
    m = jnp.max(l, axis=0, keepdims=True)
    se = jnp.sum(jnp.exp(l - m), axis=0, keepdims=True)
    lse = jnp.log(se) + m
    cio = lax.broadcasted_iota(jnp.int32, (_C, _LB), 0)
    lfree = jnp.sum(jnp.where(cio == _FREE, l, 0.0), axis=0, keepdims=True)
    lgt = jnp.sum(jnp.where(cio == gt, l, 0.0), axis=0, keepdims=True)
    p = jnp.clip(1.0 - jnp.exp(lfree - lse), _EPS, 1.0 - _EPS)
    pre = -jnp.log(1.0 - p)
    hit = -jnp.log(p) + (lse - lgt)
    occf = (gt != _FREE) & (mk > 0)
    w = jnp.where(occf, jnp.maximum(hit, 1e-30), -jnp.maximum(pre, 1e-30))
    out_ref[0] = w


def _table_call(lt, gt3, mk3):
    return pl.pallas_call(
        _table_kernel,
        grid=(_NBLK,),
        in_specs=[
            pl.BlockSpec((_C, _LB), lambda i: (0, i)),
            pl.BlockSpec((1, 1, _LB), lambda i: (i, 0, 0)),
            pl.BlockSpec((1, 1, _LB), lambda i: (i, 0, 0)),
        ],
        out_specs=pl.BlockSpec((1, 1, _LB), lambda i: (i, 0, 0)),
        out_shape=jax.ShapeDtypeStruct((_NBLK, 1, _LB), jnp.float32),
    )(lt, gt3, mk3)


@functools.cache
def _get_scan_call():
    mesh = plsc.VectorSubcoreMesh(core_axis_name="c", subcore_axis_name="s",
                                  num_cores=_NC, num_subcores=_NS)
    return functools.partial(
        pl.kernel,
        out_type=jax.ShapeDtypeStruct((_NW, 4, _LANES), jnp.float32),
        mesh=mesh,
        scratch_types=[
            pltpu.VMEM((_NCHUNK, 128), jnp.int32),
            pltpu.VMEM((_NCHUNK * 128,), jnp.float32),
            pltpu.VMEM((_RPW,), jnp.int32),
            pltpu.VMEM((4, _LANES), jnp.float32),
            pltpu.SemaphoreType.DMA,
        ],
    )(_scan_body)


def _scan_body(w_hbm, idx_hbm, e_hbm, out_hbm, idx_v, vals_v, e_v, acc_v,
               sem):
    wid = lax.axis_index("s") * _NC + lax.axis_index("c")
    pltpu.sync_copy(idx_hbm.at[wid], idx_v)
    pltpu.sync_copy(e_hbm.at[wid], e_v)

    def fire(j, carry):
        pltpu.async_copy(w_hbm.at[idx_v.at[j]],
                         vals_v.at[pl.ds(j * 128, 128)], sem)
        return carry

    lax.fori_loop(0, _NCHUNK, fire, 0, unroll=False)

    def drain(j, carry):
        pltpu.make_async_copy(w_hbm.at[idx_v.at[j]],
                              vals_v.at[pl.ds(j * 128, 128)], sem).wait()
        return carry

    lax.fori_loop(0, _NCHUNK, drain, 0, unroll=False)

    zeros = jnp.zeros((_LANES,), jnp.float32)
    a_pre, a_prec, a_hit, a_hitc = zeros, zeros, zeros, zeros
    for g in range(_GPW):
        e_g = e_v[pl.ds(g * _LANES, _LANES)]

        def body(s, c, g=g, e_g=e_g):
            hitv, pv, pc, ap, apc, ah, ahc = c
            wv = vals_v[pl.ds(g * _GSTRIDE + s * _LANES, _LANES)]
            occ = jnp.where(wv > 0.0, 1.0, 0.0)
            inr = jnp.where(s < e_g, 1.0, 0.0)
            alive = (1.0 - hitv) * inr
            take = occ * alive
            ap = ap + take * pv
            apc = apc + take * pc
            ah = ah + take * wv
            ahc = ahc + take
            hitv = hitv + take
            upd = alive * (1.0 - occ)
            pv = pv - upd * wv
            pc = pc + upd
            return (hitv, pv, pc, ap, apc, ah, ahc)

        init = (zeros, zeros, zeros, a_pre, a_prec, a_hit, a_hitc)
        res = lax.fori_loop(0, _SUNI, body, init, unroll=False)
        a_pre, a_prec, a_hit, a_hitc = res[3], res[4], res[5], res[6]

    acc_v[0] = a_pre
    acc_v[1] = a_prec
    acc_v[2] = a_hit
    acc_v[3] = a_hitc
    pltpu.sync_copy(acc_v, out_hbm.at[wid])


def kernel(occ_logits, voxel_semantics, mask_camera):
    lt = jnp.transpose(occ_logits.reshape(_B * _N, _C))
    gt3 = voxel_semantics.reshape(_NBLK, 1, _LB)
    mk3 = mask_camera.reshape(_NBLK, 1, _LB)
    tbl = _table_call(lt, gt3, mk3).reshape(_B * _N)
    w_full = jnp.concatenate(
        [tbl, jnp.full((8,), -1e-30, dtype=jnp.float32)])
    parts = _get_scan_call()(w_full, _IDX_NP, _E_NP)
    sums = jnp.sum(parts, axis=(0, 2))
    loss = (sums[0] / jnp.maximum(sums[1], 1.0)
            + sums[2] / jnp.maximum(sums[3], 1.0))
    return loss


# speedup vs baseline: 4.6694x; 1.0020x over previous
import functools
import math

import numpy as np
import jax
import jax.numpy as jnp
from jax import lax
from jax.experimental import pallas as pl
from jax.experimental.pallas import tpu as pltpu
from jax.experimental.pallas import tpu_sc as plsc

_B = 2
_X, _Y, _Z = 200, 200, 16
_N = _X * _Y * _Z
_C = 18
_FREE = 17
_EPS = 1e-06
_NUM_RAYS = 2000
_MAX_DIST = 60.0
_VOXEL = 0.4
_PC_MIN = np.array([-40.0, -40.0, -1.0], dtype=np.float32)
_ORIGIN = np.array([0.9858, 0.0, 1.8402], dtype=np.float32)
_GRID = np.array([_X, _Y, _Z], dtype=np.int32)

_NC = 2
_NS = 16
_NW = _NC * _NS
_LANES = 16
_GPW = 8
_RPW = _GPW * _LANES
_DUMMY = _B * _N


def _ray_constants():
    pitch_angles = []
    for k in range(10):
        pitch_angles.append(-(math.pi / 2 - math.atan(k + 1)))
    while pitch_angles[-1] < 0.21:
        delta = pitch_angles[-1] - pitch_angles[-2]
        pitch_angles.append(pitch_angles[-1] + delta)
    rays = []
    for pitch in pitch_angles:
        for az_deg in range(360):
            az = math.radians(az_deg)
            rays.append((math.cos(pitch) * math.cos(az),
                         math.cos(pitch) * math.sin(az),
                         math.sin(pitch)))
    rays = np.array(rays, dtype=np.float32)
    dt = (_VOXEL / (np.abs(rays).max(axis=1) + 1e-08)).astype(np.float32)
    s_max = int(min(np.ceil(_MAX_DIST / dt.min()), 512))
    sel = np.linspace(0, rays.shape[0] - 1, _NUM_RAYS).astype(np.int64)
    dirs = rays[sel]
    dts = dt[sel]
    t = (np.arange(s_max, dtype=np.float32)[None, :] + 0.5) * dts[:, None]
    pts = _ORIGIN[None, None, :] + dirs[:, None, :] * t[:, :, None]
    vox = np.floor((pts - _PC_MIN[None, None, :]) / _VOXEL).astype(np.int32)
    in_range = (np.all((vox >= 0) & (vox < _GRID[None, None, :]), axis=-1)
                & (t <= _MAX_DIST))
    voxc = np.clip(vox, 0, _GRID[None, None, :] - 1)
    flat = (voxc[..., 0] * (_Y * _Z) + voxc[..., 1] * _Z + voxc[..., 2])
    return flat.astype(np.int64), in_range, s_max


def _build_tables():
    flat, in_range, s_max = _ray_constants()
    e = in_range.sum(axis=1).astype(np.int32)
    assert np.array_equal(in_range, np.arange(s_max)[None, :] < e[:, None])
    s_uni = int(-(-int(e.max()) // 8) * 8)
    rtot = _NW * _RPW
    idx_full = np.zeros((rtot, s_uni), dtype=np.int64)
    smask = np.arange(s_uni)[None, :] < e[:, None]
    base = flat[:, :s_uni]
    for b in range(_B):
        blk = idx_full[b * _NUM_RAYS:(b + 1) * _NUM_RAYS]
        blk[smask] = (b * _N + base)[smask]
    e_full = np.zeros((rtot,), dtype=np.int32)
    e_full[:_NUM_RAYS] = e
    e_full[_NUM_RAYS:2 * _NUM_RAYS] = e
    idx_v = idx_full.reshape(_NW, _GPW, _LANES, s_uni).transpose(0, 1, 3, 2)
    n_chunk = _GPW * s_uni * _LANES // 128
    idx_chunks = np.ascontiguousarray(idx_v).reshape(_NW, n_chunk * 128)
    e_tbl = e_full.reshape(_NW, _RPW)
    return idx_chunks.astype(np.int32), e_tbl, s_uni, n_chunk


_IDX_NP, _E_NP, _SUNI, _NCHUNK = _build_tables()
_GSTRIDE = _SUNI * _LANES

_LB = 12800
_NBLK = _B * _N // _LB


def _table_kernel(lt_ref, gt_ref, mk_ref, out_ref):
    l = lt_ref[...]
    gt = gt_ref[0]
    mk = mk_ref[0]
    m = jnp.max(l, axis=0, keepdims=True)
    se = jnp.sum(jnp.exp(l - m), axis=0, keepdims=True)
    lse = jnp.log(se) + m
    cio = lax.broadcasted_iota(jnp.int32, (_C, _LB), 0)
    lfree = jnp.sum(jnp.where(cio == _FREE, l, 0.0), axis=0, keepdims=True)
    lgt = jnp.sum(jnp.where(cio == gt, l, 0.0), axis=0, keepdims=True)
    p = jnp.clip(1.0 - jnp.exp(lfree - lse), _EPS, 1.0 - _EPS)
    pre = -jnp.log(1.0 - p)
    hit = -jnp.log(p) + (lse - lgt)
    occf = (gt != _FREE) & (mk > 0)
    w = jnp.where(occf, jnp.maximum(hit, 1e-30), -jnp.maximum(pre, 1e-30))
    out_ref[0] = w


def _table_call(lt, gt3, mk3):
    return pl.pallas_call(
        _table_kernel,
        grid=(_NBLK,),
        in_specs=[
            pl.BlockSpec((_C, _LB), lambda i: (0, i)),
            pl.BlockSpec((1, 1, _LB), lambda i: (i, 0, 0)),
            pl.BlockSpec((1, 1, _LB), lambda i: (i, 0, 0)),
        ],
        out_specs=pl.BlockSpec((1, 1, _LB), lambda i: (i, 0, 0)),
        out_shape=jax.ShapeDtypeStruct((_NBLK, 1, _LB), jnp.float32),
    )(lt, gt3, mk3)


@functools.cache
def _get_scan_call():
    mesh = plsc.VectorSubcoreMesh(core_axis_name="c", subcore_axis_name="s",
                                  num_cores=_NC, num_subcores=_NS)
    return functools.partial(
        pl.kernel,
        out_type=jax.ShapeDtypeStruct((_NW, 4, _LANES), jnp.float32),
        mesh=mesh,
        scratch_types=[
            pltpu.VMEM((_NCHUNK * 128,), jnp.int32),
            pltpu.VMEM((_NCHUNK * 128,), jnp.float32),
            pltpu.VMEM((_RPW,), jnp.int32),
            pltpu.VMEM((4, _LANES), jnp.float32),
            pltpu.SemaphoreType.DMA,
        ],
    )(_scan_body)


def _scan_body(w_hbm, idx_hbm, e_hbm, out_hbm, idx_v, vals_v, e_v, acc_v,
               sem):
    wid = lax.axis_index("s") * _NC + lax.axis_index("c")
    pltpu.sync_copy(idx_hbm.at[wid], idx_v)
    pltpu.sync_copy(e_hbm.at[wid], e_v)
    pltpu.async_copy(w_hbm.at[idx_v], vals_v, sem).wait()

    zeros = jnp.zeros((_LANES,), jnp.float32)
    a_pre, a_prec, a_hit, a_hitc = zeros, zeros, zeros, zeros
    for g in range(_GPW):
        e_g = e_v[pl.ds(g * _LANES, _LANES)]

        def body(s, c, g=g, e_g=e_g):
            hitv, pv, pc, ap, apc, ah, ahc = c
            wv = vals_v[pl.ds(g * _GSTRIDE + s * _LANES, _LANES)]
            occ = jnp.where(wv > 0.0, 1.0, 0.0)
            inr = jnp.where(s < e_g, 1.0, 0.0)
            alive = (1.0 - hitv) * inr
            take = occ * alive
            ap = ap + take * pv
            apc = apc + take * pc
            ah = ah + take * wv
            ahc = ahc + take
            hitv = hitv + take
            upd = alive * (1.0 - occ)
            pv = pv - upd * wv
            pc = pc + upd
            return (hitv, pv, pc, ap, apc, ah, ahc)

        init = (zeros, zeros, zeros, a_pre, a_prec, a_hit, a_hitc)
        res = lax.fori_loop(0, _SUNI, body, init, unroll=False)
        a_pre, a_prec, a_hit, a_hitc = res[3], res[4], res[5], res[6]

    acc_v[0] = a_pre
    acc_v[1] = a_prec
    acc_v[2] = a_hit
    acc_v[3] = a_hitc
    pltpu.sync_copy(acc_v, out_hbm.at[wid])


def kernel(occ_logits, voxel_semantics, mask_camera):
    lt = jnp.transpose(occ_logits.reshape(_B * _N, _C))
    gt3 = voxel_semantics.reshape(_NBLK, 1, _LB)
    mk3 = mask_camera.reshape(_NBLK, 1, _LB)
    w_full = _table_call(lt, gt3, mk3).reshape(_B * _N)
    parts = _get_scan_call()(w_full, _IDX_NP, _E_NP)
    sums = jnp.sum(parts, axis=(0, 2))
    loss = (sums[0] / jnp.maximum(sums[1], 1.0)
            + sums[2] / jnp.maximum(sums[3], 1.0))
    return loss


# speedup vs baseline: 5.8685x; 1.2568x over previous
import functools
import math

import numpy as np
import jax
import jax.numpy as jnp
from jax import lax
from jax.experimental import pallas as pl
from jax.experimental.pallas import tpu as pltpu
from jax.experimental.pallas import tpu_sc as plsc

_B = 2
_X, _Y, _Z = 200, 200, 16
_N = _X * _Y * _Z
_C = 18
_FREE = 17
_EPS = 1e-06
_NUM_RAYS = 2000
_MAX_DIST = 60.0
_VOXEL = 0.4
_PC_MIN = np.array([-40.0, -40.0, -1.0], dtype=np.float32)
_ORIGIN = np.array([0.9858, 0.0, 1.8402], dtype=np.float32)
_GRID = np.array([_X, _Y, _Z], dtype=np.int32)

_NC = 2
_NS = 16
_NW = _NC * _NS
_LANES = 16
_GPW = 8
_RPW = _GPW * _LANES
_DUMMY = _B * _N


def _ray_constants():
    pitch_angles = []
    for k in range(10):
        pitch_angles.append(-(math.pi / 2 - math.atan(k + 1)))
    while pitch_angles[-1] < 0.21:
        delta = pitch_angles[-1] - pitch_angles[-2]
        pitch_angles.append(pitch_angles[-1] + delta)
    rays = []
    for pitch in pitch_angles:
        for az_deg in range(360):
            az = math.radians(az_deg)
            rays.append((math.cos(pitch) * math.cos(az),
                         math.cos(pitch) * math.sin(az),
                         math.sin(pitch)))
    rays = np.array(rays, dtype=np.float32)
    dt = (_VOXEL / (np.abs(rays).max(axis=1) + 1e-08)).astype(np.float32)
    s_max = int(min(np.ceil(_MAX_DIST / dt.min()), 512))
    sel = np.linspace(0, rays.shape[0] - 1, _NUM_RAYS).astype(np.int64)
    dirs = rays[sel]
    dts = dt[sel]
    t = (np.arange(s_max, dtype=np.float32)[None, :] + 0.5) * dts[:, None]
    pts = _ORIGIN[None, None, :] + dirs[:, None, :] * t[:, :, None]
    vox = np.floor((pts - _PC_MIN[None, None, :]) / _VOXEL).astype(np.int32)
    in_range = (np.all((vox >= 0) & (vox < _GRID[None, None, :]), axis=-1)
                & (t <= _MAX_DIST))
    voxc = np.clip(vox, 0, _GRID[None, None, :] - 1)
    flat = (voxc[..., 0] * (_Y * _Z) + voxc[..., 1] * _Z + voxc[..., 2])
    return flat.astype(np.int64), in_range, s_max


def _build_tables():
    flat, in_range, s_max = _ray_constants()
    e = in_range.sum(axis=1).astype(np.int32)
    assert np.array_equal(in_range, np.arange(s_max)[None, :] < e[:, None])
    s_uni = int(-(-int(e.max()) // 8) * 8)
    rtot = _NW * _RPW
    idx_full = np.zeros((rtot, s_uni), dtype=np.int64)
    base = flat[:, :s_uni]
    for b in range(_B):
        idx_full[b * _NUM_RAYS:(b + 1) * _NUM_RAYS] = b * _N + base
    npad = rtot - _B * _NUM_RAYS
    idx_full[_B * _NUM_RAYS:] = idx_full[:npad]
    e_full = np.zeros((rtot,), dtype=np.int32)
    e_full[:_NUM_RAYS] = e
    e_full[_NUM_RAYS:2 * _NUM_RAYS] = e
    idx_v = idx_full.reshape(_NW, _GPW, _LANES, s_uni).transpose(0, 1, 3, 2)
    n_chunk = _GPW * s_uni * _LANES // 128
    idx_chunks = np.ascontiguousarray(idx_v).reshape(_NW, n_chunk * 128)
    e_tbl = e_full.reshape(_NW, _RPW)
    return idx_chunks.astype(np.int32), e_tbl, s_uni, n_chunk


_IDX_NP, _E_NP, _SUNI, _NCHUNK = _build_tables()
_GSTRIDE = _SUNI * _LANES

_LB = 12800
_NBLK = _B * _N // _LB


def _table_kernel(lt_ref, gt_ref, mk_ref, out_ref):
    l = lt_ref[...]
    gt = gt_ref[0]
    mk = mk_ref[0]
    m = jnp.max(l, axis=0, keepdims=True)
    se = jnp.sum(jnp.exp(l - m), axis=0, keepdims=True)
    lse = jnp.log(se) + m
    cio = lax.broadcasted_iota(jnp.int32, (_C, _LB), 0)
    lfree = jnp.sum(jnp.where(cio == _FREE, l, 0.0), axis=0, keepdims=True)
    lgt = jnp.sum(jnp.where(cio == gt, l, 0.0), axis=0, keepdims=True)
    p = jnp.clip(1.0 - jnp.exp(lfree - lse), _EPS, 1.0 - _EPS)
    pre = -jnp.log(1.0 - p)
    hit = -jnp.log(p) + (lse - lgt)
    occf = (gt != _FREE) & (mk > 0)
    w = jnp.where(occf, jnp.maximum(hit, 1e-30), -jnp.maximum(pre, 1e-30))
    out_ref[0] = w


def _table_call(lt, gt3, mk3):
    return pl.pallas_call(
        _table_kernel,
        grid=(_NBLK,),
        in_specs=[
            pl.BlockSpec((_C, _LB), lambda i: (0, i)),
            pl.BlockSpec((1, 1, _LB), lambda i: (i, 0, 0)),
            pl.BlockSpec((1, 1, _LB), lambda i: (i, 0, 0)),
        ],
        out_specs=pl.BlockSpec((1, 1, _LB), lambda i: (i, 0, 0)),
        out_shape=jax.ShapeDtypeStruct((_NBLK, 1, _LB), jnp.float32),
    )(lt, gt3, mk3)


@functools.cache
def _get_scan_call():
    mesh = plsc.VectorSubcoreMesh(core_axis_name="c", subcore_axis_name="s",
                                  num_cores=_NC, num_subcores=_NS)
    return functools.partial(
        pl.kernel,
        out_type=jax.ShapeDtypeStruct((_NW, 4, _LANES), jnp.float32),
        mesh=mesh,
        scratch_types=[
            pltpu.VMEM((_NCHUNK * 128,), jnp.int32),
            pltpu.VMEM((_NCHUNK * 128,), jnp.float32),
            pltpu.VMEM((_RPW,), jnp.int32),
            pltpu.VMEM((4, _LANES), jnp.float32),
            pltpu.SemaphoreType.DMA,
        ],
    )(_scan_body)


def _scan_body(w_hbm, idx_hbm, e_hbm, out_hbm, idx_v, vals_v, e_v, acc_v,
               sem):
    wid = lax.axis_index("s") * _NC + lax.axis_index("c")
    pltpu.sync_copy(idx_hbm.at[wid], idx_v)
    pltpu.sync_copy(e_hbm.at[wid], e_v)
    pltpu.async_copy(w_hbm.at[idx_v], vals_v, sem).wait()

    zeros = jnp.zeros((_LANES,), jnp.float32)
    a_pre, a_prec, a_hit, a_hitc = zeros, zeros, zeros, zeros
    for g in range(_GPW):
        e_g = e_v[pl.ds(g * _LANES, _LANES)]

        def body(s, c, g=g, e_g=e_g):
            hitv, pv, pc, ap, apc, ah, ahc = c
            wv = vals_v[pl.ds(g * _GSTRIDE + s * _LANES, _LANES)]
            occ = jnp.where(wv > 0.0, 1.0, 0.0)
            inr = jnp.where(s < e_g, 1.0, 0.0)
            alive = (1.0 - hitv) * inr
            take = occ * alive
            ap = ap + take * pv
            apc = apc + take * pc
            ah = ah + take * wv
            ahc = ahc + take
            hitv = hitv + take
            upd = alive * (1.0 - occ)
            pv = pv - upd * wv
            pc = pc + upd
            return (hitv, pv, pc, ap, apc, ah, ahc)

        init = (zeros, zeros, zeros, a_pre, a_prec, a_hit, a_hitc)
        res = lax.fori_loop(0, _SUNI, body, init, unroll=False)
        a_pre, a_prec, a_hit, a_hitc = res[3], res[4], res[5], res[6]

    acc_v[0] = a_pre
    acc_v[1] = a_prec
    acc_v[2] = a_hit
    acc_v[3] = a_hitc
    pltpu.sync_copy(acc_v, out_hbm.at[wid])


def kernel(occ_logits, voxel_semantics, mask_camera):
    lt = jnp.transpose(occ_logits.reshape(_B * _N, _C))
    gt3 = voxel_semantics.reshape(_NBLK, 1, _LB)
    mk3 = mask_camera.reshape(_NBLK, 1, _LB)
    w_full = _table_call(lt, gt3, mk3).reshape(_B * _N)
    parts = _get_scan_call()(w_full, _IDX_NP, _E_NP)
    sums = jnp.sum(parts, axis=(0, 2))
    loss = (sums[0] / jnp.maximum(sums[1], 1.0)
            + sums[2] / jnp.maximum(sums[3], 1.0))
    return loss


# speedup vs baseline: 12.0621x; 2.0554x over previous
import functools
import math

import numpy as np
import jax
import jax.numpy as jnp
from jax import lax
from jax.experimental import pallas as pl
from jax.experimental.pallas import tpu as pltpu
from jax.experimental.pallas import tpu_sc as plsc

_B = 2
_X, _Y, _Z = 200, 200, 16
_N = _X * _Y * _Z
_C = 18
_FREE = 17
_EPS = 1e-06
_NUM_RAYS = 2000
_MAX_DIST = 60.0
_VOXEL = 0.4
_PC_MIN = np.array([-40.0, -40.0, -1.0], dtype=np.float32)
_ORIGIN = np.array([0.9858, 0.0, 1.8402], dtype=np.float32)
_GRID = np.array([_X, _Y, _Z], dtype=np.int32)

_NC = 2
_NS = 16
_NW = _NC * _NS
_LANES = 16
_GPW = 8
_RPW = _GPW * _LANES
_DUMMY = _B * _N


def _ray_constants():
    pitch_angles = []
    for k in range(10):
        pitch_angles.append(-(math.pi / 2 - math.atan(k + 1)))
    while pitch_angles[-1] < 0.21:
        delta = pitch_angles[-1] - pitch_angles[-2]
        pitch_angles.append(pitch_angles[-1] + delta)
    rays = []
    for pitch in pitch_angles:
        for az_deg in range(360):
            az = math.radians(az_deg)
            rays.append((math.cos(pitch) * math.cos(az),
                         math.cos(pitch) * math.sin(az),
                         math.sin(pitch)))
    rays = np.array(rays, dtype=np.float32)
    dt = (_VOXEL / (np.abs(rays).max(axis=1) + 1e-08)).astype(np.float32)
    s_max = int(min(np.ceil(_MAX_DIST / dt.min()), 512))
    sel = np.linspace(0, rays.shape[0] - 1, _NUM_RAYS).astype(np.int64)
    dirs = rays[sel]
    dts = dt[sel]
    t = (np.arange(s_max, dtype=np.float32)[None, :] + 0.5) * dts[:, None]
    pts = _ORIGIN[None, None, :] + dirs[:, None, :] * t[:, :, None]
    vox = np.floor((pts - _PC_MIN[None, None, :]) / _VOXEL).astype(np.int32)
    in_range = (np.all((vox >= 0) & (vox < _GRID[None, None, :]), axis=-1)
                & (t <= _MAX_DIST))
    voxc = np.clip(vox, 0, _GRID[None, None, :] - 1)
    flat = (voxc[..., 0] * (_Y * _Z) + voxc[..., 1] * _Z + voxc[..., 2])
    return flat.astype(np.int64), in_range, s_max


def _build_tables():
    flat, in_range, s_max = _ray_constants()
    e = in_range.sum(axis=1).astype(np.int32)
    assert np.array_equal(in_range, np.arange(s_max)[None, :] < e[:, None])
    s_uni = int(-(-int(e.max()) // 8) * 8)
    rtot = _NW * _RPW
    idx_full = np.zeros((rtot, s_uni), dtype=np.int64)
    base = flat[:, :s_uni]
    for b in range(_B):
        idx_full[b * _NUM_RAYS:(b + 1) * _NUM_RAYS] = b * _N + base
    npad = rtot - _B * _NUM_RAYS
    idx_full[_B * _NUM_RAYS:] = idx_full[:npad]
    e_full = np.zeros((rtot,), dtype=np.int32)
    e_full[:_NUM_RAYS] = e
    e_full[_NUM_RAYS:2 * _NUM_RAYS] = e
    idx_v = idx_full.reshape(_NW, _GPW, _LANES, s_uni).transpose(0, 1, 3, 2)
    n_chunk = _GPW * s_uni * _LANES // 128
    idx_chunks = np.ascontiguousarray(idx_v).reshape(_NW, n_chunk * 128)
    e_tbl = e_full.reshape(_NW, _RPW)
    return idx_chunks.astype(np.int32), e_tbl, s_uni, n_chunk


_IDX_NP, _E_NP, _SUNI, _NCHUNK = _build_tables()
_GSTRIDE = _SUNI * _LANES

_RB = 2048
_NBLK = _B * _N // _RB


def _table_kernel(l_ref, gtc_ref, gtl_ref, mkl_ref, out_ref):
    l = l_ref[0]
    gtc = gtc_ref[0]
    gtl = gtl_ref[0]
    mkl = mkl_ref[0]
    cio = lax.broadcasted_iota(jnp.int32, (_RB, _C), 1)
    lsel = jnp.where(cio == gtc, l, 0.0)
    ex = jnp.exp(l)
    ones = jnp.ones((1, _C), jnp.float32)
    efree = (lax.broadcasted_iota(jnp.int32, (1, _C), 1) == _FREE
             ).astype(jnp.float32)
    dn = (((1,), (1,)), ((), ()))
    se = lax.dot_general(ones, ex, dn)
    lgt = lax.dot_general(ones, lsel, dn)
    lfree = lax.dot_general(efree, l, dn)
    lse = jnp.log(se)
    p = jnp.clip(1.0 - jnp.exp(lfree - lse), _EPS, 1.0 - _EPS)
    pre = -jnp.log(1.0 - p)
    hit = -jnp.log(p) + (lse - lgt)
    occf = (gtl != _FREE) & (mkl > 0)
    w = jnp.where(occf, jnp.maximum(hit, 1e-30), -jnp.maximum(pre, 1e-30))
    out_ref[0] = w


def _table_call(l3, gtc3, gtl3, mkl3):
    return pl.pallas_call(
        _table_kernel,
        grid=(_NBLK,),
        in_specs=[
            pl.BlockSpec((1, _RB, _C), lambda i: (i, 0, 0)),
            pl.BlockSpec((1, _RB, 1), lambda i: (i, 0, 0)),
            pl.BlockSpec((1, 1, _RB), lambda i: (i, 0, 0)),
            pl.BlockSpec((1, 1, _RB), lambda i: (i, 0, 0)),
        ],
        out_specs=pl.BlockSpec((1, 1, _RB), lambda i: (i, 0, 0)),
        out_shape=jax.ShapeDtypeStruct((_NBLK, 1, _RB), jnp.float32),
    )(l3, gtc3, gtl3, mkl3)


@functools.cache
def _get_scan_call():
    mesh = plsc.VectorSubcoreMesh(core_axis_name="c", subcore_axis_name="s",
                                  num_cores=_NC, num_subcores=_NS)
    return functools.partial(
        pl.kernel,
        out_type=jax.ShapeDtypeStruct((_NW, 4, _LANES), jnp.float32),
        mesh=mesh,
        scratch_types=[
            pltpu.VMEM((_NCHUNK * 128,), jnp.int32),
            pltpu.VMEM((_NCHUNK * 128,), jnp.float32),
            pltpu.VMEM((_RPW,), jnp.int32),
            pltpu.VMEM((4, _LANES), jnp.float32),
            pltpu.SemaphoreType.DMA,
        ],
    )(_scan_body)


def _scan_body(w_hbm, idx_hbm, e_hbm, out_hbm, idx_v, vals_v, e_v, acc_v,
               sem):
    wid = lax.axis_index("s") * _NC + lax.axis_index("c")
    pltpu.sync_copy(idx_hbm.at[wid], idx_v)
    pltpu.sync_copy(e_hbm.at[wid], e_v)
    pltpu.async_copy(w_hbm.at[idx_v], vals_v, sem).wait()

    zeros = jnp.zeros((_LANES,), jnp.float32)
    a_pre, a_prec, a_hit, a_hitc = zeros, zeros, zeros, zeros
    for g in range(_GPW):
        e_g = e_v[pl.ds(g * _LANES, _LANES)]

        def body(s, c, g=g, e_g=e_g):
            hitv, pv, pc, ap, apc, ah, ahc = c
            wv = vals_v[pl.ds(g * _GSTRIDE + s * _LANES, _LANES)]
            occ = jnp.where(wv > 0.0, 1.0, 0.0)
            inr = jnp.where(s < e_g, 1.0, 0.0)
            alive = (1.0 - hitv) * inr
            take = occ * alive
            ap = ap + take * pv
            apc = apc + take * pc
            ah = ah + take * wv
            ahc = ahc + take
            hitv = hitv + take
            upd = alive * (1.0 - occ)
            pv = pv - upd * wv
            pc = pc + upd
            return (hitv, pv, pc, ap, apc, ah, ahc)

        init = (zeros, zeros, zeros, a_pre, a_prec, a_hit, a_hitc)
        res = lax.fori_loop(0, _SUNI, body, init, unroll=False)
        a_pre, a_prec, a_hit, a_hitc = res[3], res[4], res[5], res[6]

    acc_v[0] = a_pre
    acc_v[1] = a_prec
    acc_v[2] = a_hit
    acc_v[3] = a_hitc
    pltpu.sync_copy(acc_v, out_hbm.at[wid])


def kernel(occ_logits, voxel_semantics, mask_camera):
    l3 = occ_logits.reshape(_NBLK, _RB, _C)
    gtc3 = voxel_semantics.reshape(_NBLK, _RB, 1)
    gtl3 = voxel_semantics.reshape(_NBLK, 1, _RB)
    mkl3 = mask_camera.reshape(_NBLK, 1, _RB)
    w_full = _table_call(l3, gtc3, gtl3, mkl3).reshape(_B * _N)
    parts = _get_scan_call()(w_full, _IDX_NP, _E_NP)
    sums = jnp.sum(parts, axis=(0, 2))
    loss = (sums[0] / jnp.maximum(sums[1], 1.0)
            + sums[2] / jnp.maximum(sums[3], 1.0))
    return loss


# speedup vs baseline: 14.0987x; 1.1688x over previous
import functools
import math

import numpy as np
import jax
import jax.numpy as jnp
from jax import lax
from jax.experimental import pallas as pl
from jax.experimental.pallas import tpu as pltpu
from jax.experimental.pallas import tpu_sc as plsc

_B = 2
_X, _Y, _Z = 200, 200, 16
_N = _X * _Y * _Z
_C = 18
_FREE = 17
_EPS = 1e-06
_NUM_RAYS = 2000
_MAX_DIST = 60.0
_VOXEL = 0.4
_PC_MIN = np.array([-40.0, -40.0, -1.0], dtype=np.float32)
_ORIGIN = np.array([0.9858, 0.0, 1.8402], dtype=np.float32)
_GRID = np.array([_X, _Y, _Z], dtype=np.int32)

_NC = 2
_NS = 16
_NW = _NC * _NS
_LANES = 16
_GPW = 8
_RPW = _GPW * _LANES
_DUMMY = _B * _N


def _ray_constants():
    pitch_angles = []
    for k in range(10):
        pitch_angles.append(-(math.pi / 2 - math.atan(k + 1)))
    while pitch_angles[-1] < 0.21:
        delta = pitch_angles[-1] - pitch_angles[-2]
        pitch_angles.append(pitch_angles[-1] + delta)
    rays = []
    for pitch in pitch_angles:
        for az_deg in range(360):
            az = math.radians(az_deg)
            rays.append((math.cos(pitch) * math.cos(az),
                         math.cos(pitch) * math.sin(az),
                         math.sin(pitch)))
    rays = np.array(rays, dtype=np.float32)
    dt = (_VOXEL / (np.abs(rays).max(axis=1) + 1e-08)).astype(np.float32)
    s_max = int(min(np.ceil(_MAX_DIST / dt.min()), 512))
    sel = np.linspace(0, rays.shape[0] - 1, _NUM_RAYS).astype(np.int64)
    dirs = rays[sel]
    dts = dt[sel]
    t = (np.arange(s_max, dtype=np.float32)[None, :] + 0.5) * dts[:, None]
    pts = _ORIGIN[None, None, :] + dirs[:, None, :] * t[:, :, None]
    vox = np.floor((pts - _PC_MIN[None, None, :]) / _VOXEL).astype(np.int32)
    in_range = (np.all((vox >= 0) & (vox < _GRID[None, None, :]), axis=-1)
                & (t <= _MAX_DIST))
    voxc = np.clip(vox, 0, _GRID[None, None, :] - 1)
    flat = (voxc[..., 0] * (_Y * _Z) + voxc[..., 1] * _Z + voxc[..., 2])
    return flat.astype(np.int64), in_range, s_max


def _build_tables():
    flat, in_range, s_max = _ray_constants()
    e = in_range.sum(axis=1).astype(np.int32)
    assert np.array_equal(in_range, np.arange(s_max)[None, :] < e[:, None])
    s_uni = int(-(-int(e.max()) // 8) * 8)
    rtot = _NW * _RPW
    idx_full = np.zeros((rtot, s_uni), dtype=np.int64)
    base = flat[:, :s_uni]
    for b in range(_B):
        idx_full[b * _NUM_RAYS:(b + 1) * _NUM_RAYS] = b * _N + base
    npad = rtot - _B * _NUM_RAYS
    idx_full[_B * _NUM_RAYS:] = idx_full[:npad]
    e_full = np.zeros((rtot,), dtype=np.int32)
    e_full[:_NUM_RAYS] = e
    e_full[_NUM_RAYS:2 * _NUM_RAYS] = e
    idx_v = idx_full.reshape(_NW, _GPW, _LANES, s_uni).transpose(0, 1, 3, 2)
    n_chunk = _GPW * s_uni * _LANES // 128
    idx_chunks = np.ascontiguousarray(idx_v).reshape(_NW, n_chunk * 128)
    e_tbl = e_full.reshape(_NW, _RPW)
    return idx_chunks.astype(np.int32), e_tbl, s_uni, n_chunk


_IDX_NP, _E_NP, _SUNI, _NCHUNK = _build_tables()
_GSTRIDE = _SUNI * _LANES

_VSUP = 64
_SROW = _VSUP * _C
_NSB = 160
_NBLK = _B * _N // (_NSB * _VSUP)


def _table_kernel(l_ref, gt_ref, mk_ref, out_ref):
    lf = l_ref[0]
    gt = gt_ref[0]
    mk = mk_ref[0]
    jj = lax.broadcasted_iota(jnp.int32, (_SROW, _VSUP), 0)
    vv = lax.broadcasted_iota(jnp.int32, (_SROW, _VSUP), 1)
    own = (jj // _C) == vv
    st = jnp.where(own, 1.0, 0.0)
    stf = jnp.where(own & (jj % _C == _FREE), 1.0, 0.0)
    dn = (((1,), (0,)), ((), ()))
    ex = jnp.exp(lf)
    se = lax.dot_general(ex, st, dn)
    lfree = lax.dot_general(lf, stf, dn)
    gtf = gt.astype(jnp.float32)
    ee = lax.dot_general(gtf, st, (((1,), (1,)), ((), ())))
    cio = lax.broadcasted_iota(jnp.int32, (_NSB, _SROW), 1) % _C
    lsel = jnp.where(cio.astype(jnp.float32) == ee, lf, 0.0)
    lgt = lax.dot_general(lsel, st, dn)
    lse = jnp.log(se)
    p = jnp.clip(1.0 - jnp.exp(lfree - lse), _EPS, 1.0 - _EPS)
    pre = -jnp.log(1.0 - p)
    hit = -jnp.log(p) + (lse - lgt)
    occf = (gt != _FREE) & (mk > 0)
    w = jnp.where(occf, jnp.maximum(hit, 1e-30), -jnp.maximum(pre, 1e-30))
    out_ref[0] = w


def _table_call(l3, gt3, mk3):
    return pl.pallas_call(
        _table_kernel,
        grid=(_NBLK,),
        in_specs=[
            pl.BlockSpec((1, _NSB, _SROW), lambda i: (i, 0, 0)),
            pl.BlockSpec((1, _NSB, _VSUP), lambda i: (i, 0, 0)),
            pl.BlockSpec((1, _NSB, _VSUP), lambda i: (i, 0, 0)),
        ],
        out_specs=pl.BlockSpec((1, _NSB, _VSUP), lambda i: (i, 0, 0)),
        out_shape=jax.ShapeDtypeStruct((_NBLK, _NSB, _VSUP), jnp.float32),
    )(l3, gt3, mk3)


@functools.cache
def _get_scan_call():
    mesh = plsc.VectorSubcoreMesh(core_axis_name="c", subcore_axis_name="s",
                                  num_cores=_NC, num_subcores=_NS)
    return functools.partial(
        pl.kernel,
        out_type=jax.ShapeDtypeStruct((_NW, 4, _LANES), jnp.float32),
        mesh=mesh,
        scratch_types=[
            pltpu.VMEM((_NCHUNK * 128,), jnp.int32),
            pltpu.VMEM((_NCHUNK * 128,), jnp.float32),
            pltpu.VMEM((_RPW,), jnp.int32),
            pltpu.VMEM((4, _LANES), jnp.float32),
            pltpu.SemaphoreType.DMA,
        ],
    )(_scan_body)


def _scan_body(w_hbm, idx_hbm, e_hbm, out_hbm, idx_v, vals_v, e_v, acc_v,
               sem):
    wid = lax.axis_index("s") * _NC + lax.axis_index("c")
    pltpu.sync_copy(idx_hbm.at[wid], idx_v)
    pltpu.sync_copy(e_hbm.at[wid], e_v)
    pltpu.async_copy(w_hbm.at[idx_v], vals_v, sem).wait()

    zeros = jnp.zeros((_LANES,), jnp.float32)
    a_pre, a_prec, a_hit, a_hitc = zeros, zeros, zeros, zeros
    for g in range(_GPW):
        e_g = e_v[pl.ds(g * _LANES, _LANES)]

        def body(s, c, g=g, e_g=e_g):
            hitv, pv, pc, ap, apc, ah, ahc = c
            wv = vals_v[pl.ds(g * _GSTRIDE + s * _LANES, _LANES)]
            occ = jnp.where(wv > 0.0, 1.0, 0.0)
            inr = jnp.where(s < e_g, 1.0, 0.0)
            alive = (1.0 - hitv) * inr
            take = occ * alive
            ap = ap + take * pv
            apc = apc + take * pc
            ah = ah + take * wv
            ahc = ahc + take
            hitv = hitv + take
            upd = alive * (1.0 - occ)
            pv = pv - upd * wv
            pc = pc + upd
            return (hitv, pv, pc, ap, apc, ah, ahc)

        init = (zeros, zeros, zeros, a_pre, a_prec, a_hit, a_hitc)
        res = lax.fori_loop(0, _SUNI, body, init, unroll=False)
        a_pre, a_prec, a_hit, a_hitc = res[3], res[4], res[5], res[6]

    acc_v[0] = a_pre
    acc_v[1] = a_prec
    acc_v[2] = a_hit
    acc_v[3] = a_hitc
    pltpu.sync_copy(acc_v, out_hbm.at[wid])


def kernel(occ_logits, voxel_semantics, mask_camera):
    l3 = occ_logits.reshape(_NBLK, _NSB, _SROW)
    gt3 = voxel_semantics.reshape(_NBLK, _NSB, _VSUP)
    mk3 = mask_camera.reshape(_NBLK, _NSB, _VSUP)
    w_full = _table_call(l3, gt3, mk3).reshape(_B * _N)
    parts = _get_scan_call()(w_full, _IDX_NP, _E_NP)
    sums = jnp.sum(parts, axis=(0, 2))
    loss = (sums[0] / jnp.maximum(sums[1], 1.0)
            + sums[2] / jnp.maximum(sums[3], 1.0))
    return loss
